# Initial kernel scaffold; baseline (speedup 1.0000x reference)
import functools

import jax
import jax.numpy as jnp
from jax import lax
from jax.experimental import pallas as pl
from jax.experimental.pallas import tpu as pltpu
from jax.experimental.pallas import tpu_sc as plsc

N = 10000
D = 128
E = 320000
L = 3

NC = 2
NS = 16
NW = NC * NS
EPW = E // NW
CHUNK = 80
NCHUNK = EPW // CHUNK
RPT = N // NS


def _sc_aggregate(h, e3, zeros):
    mesh = plsc.VectorSubcoreMesh(core_axis_name="c", subcore_axis_name="s")

    @functools.partial(
        pl.kernel,
        out_type=jax.ShapeDtypeStruct((NC, N, D), jnp.float32),
        mesh=mesh,
        scratch_types=[
            pltpu.VMEM_SHARED((N, D), jnp.float32),
            pltpu.VMEM((NCHUNK, CHUNK), jnp.int32),
            pltpu.VMEM((NCHUNK, CHUNK), jnp.int32),
            pltpu.VMEM((CHUNK, D), jnp.float32),
            pltpu.SemaphoreType.DMA,
        ],
    )
    def agg(h_hbm, e_hbm, z_hbm, out_hbm, aggr_sh, idx_src, idx_dst, rows, sem):
        c = lax.axis_index("c")
        s = lax.axis_index("s")
        wid = c * NS + s
        pltpu.sync_copy(z_hbm.at[pl.ds(s * RPT, RPT)],
                        aggr_sh.at[pl.ds(s * RPT, RPT)])
        pltpu.sync_copy(e_hbm.at[0, pl.ds(wid * NCHUNK, NCHUNK)], idx_src)
        pltpu.sync_copy(e_hbm.at[1, pl.ds(wid * NCHUNK, NCHUNK)], idx_dst)
        plsc.subcore_barrier()

        def body(i, carry):
            pltpu.async_copy(h_hbm.at[idx_src.at[i]], rows, sem).wait()
            pltpu.sync_copy(rows, aggr_sh.at[idx_dst.at[i]], add=True)
            return carry

        lax.fori_loop(0, NCHUNK, body, 0)
        plsc.subcore_barrier()
        pltpu.sync_copy(aggr_sh.at[pl.ds(s * RPT, RPT)],
                        out_hbm.at[c, pl.ds(s * RPT, RPT)])

    return agg(h, e3, zeros)


def _mlp_body(relu_out, h_ref, p_ref, w1_ref, b1_ref, g1_ref, be1_ref,
              w2_ref, b2_ref, g2_ref, be2_ref, o_ref):
    out = h_ref[...] + p_ref[0] + p_ref[1]
    z = jnp.dot(out, w1_ref[...], preferred_element_type=jnp.float32)
    z = z + b1_ref[...]
    m = jnp.mean(z, axis=0, keepdims=True)
    v = jnp.mean((z - m) ** 2, axis=0, keepdims=True)
    z = (z - m) * lax.rsqrt(v + 1e-5) * g1_ref[...] + be1_ref[...]
    z = jnp.maximum(z, 0.0)
    z = jnp.dot(z, w2_ref[...], preferred_element_type=jnp.float32)
    z = z + b2_ref[...]
    m2 = jnp.mean(z, axis=0, keepdims=True)
    v2 = jnp.mean((z - m2) ** 2, axis=0, keepdims=True)
    z = (z - m2) * lax.rsqrt(v2 + 1e-5) * g2_ref[...] + be2_ref[...]
    if relu_out:
        z = jnp.maximum(z, 0.0)
    o_ref[...] = z


def _mlp(h, parts, w1, b1, g1, be1, w2, b2, g2, be2, relu_out):
    return pl.pallas_call(
        functools.partial(_mlp_body, relu_out),
        out_shape=jax.ShapeDtypeStruct((N, D), jnp.float32),
    )(h, parts, w1, b1.reshape(1, D), g1.reshape(1, D), be1.reshape(1, D),
      w2, b2.reshape(1, D), g2.reshape(1, D), be2.reshape(1, D))


def kernel(x, edge_index, W1, b1, g1, be1, W2, b2, g2, be2):
    e3 = edge_index.astype(jnp.int32).reshape(2, NW * NCHUNK, CHUNK)
    zeros = jnp.zeros((N, D), jnp.float32)
    h = x
    for l in range(L):
        parts = _sc_aggregate(h, e3, zeros)
        h = _mlp(h, parts, W1[l], b1[l], g1[l], be1[l],
                 W2[l], b2[l], g2[l], be2[l], l < L - 1)
    return h


# speedup vs baseline: 7.5601x; 7.5601x over previous
import functools

import jax
import jax.numpy as jnp
from jax import lax
from jax.experimental import pallas as pl
from jax.experimental.pallas import tpu as pltpu
from jax.experimental.pallas import tpu_sc as plsc

N = 10000
D = 128
E = 320000
L = 3

NC = 2
NS = 16
NW = NC * NS
EPW = E // NW
CHUNK = 125
NCHUNK = EPW // CHUNK
RPT = 624
TAIL = N - NS * RPT


def _sc_aggregate(h, e3, zeros):
    mesh = plsc.VectorSubcoreMesh(core_axis_name="c", subcore_axis_name="s")

    @functools.partial(
        pl.kernel,
        out_type=jax.ShapeDtypeStruct((NC, N, D), jnp.float32),
        mesh=mesh,
        scratch_types=[
            pltpu.VMEM_SHARED((N, D), jnp.float32),
            pltpu.VMEM((NCHUNK, CHUNK), jnp.int32),
            pltpu.VMEM((NCHUNK, CHUNK), jnp.int32),
            pltpu.VMEM((CHUNK, D), jnp.float32),
            pltpu.SemaphoreType.DMA,
        ],
    )
    def agg(h_hbm, e_hbm, z_hbm, out_hbm, aggr_sh, idx_src, idx_dst, rows, sem):
        c = lax.axis_index("c")
        s = lax.axis_index("s")
        wid = c * NS + s
        pltpu.sync_copy(z_hbm.at[pl.ds(s * RPT, RPT)],
                        aggr_sh.at[pl.ds(s * RPT, RPT)])

        @pl.when(s == NS - 1)
        def _():
            pltpu.sync_copy(z_hbm.at[pl.ds(NS * RPT, TAIL)],
                            aggr_sh.at[pl.ds(NS * RPT, TAIL)])
        pltpu.sync_copy(e_hbm.at[0, pl.ds(wid * NCHUNK, NCHUNK)], idx_src)
        pltpu.sync_copy(e_hbm.at[1, pl.ds(wid * NCHUNK, NCHUNK)], idx_dst)
        plsc.subcore_barrier()

        def body(i, carry):
            pltpu.async_copy(h_hbm.at[idx_src.at[i]], rows, sem).wait()
            pltpu.sync_copy(rows, aggr_sh.at[idx_dst.at[i]], add=True)
            return carry

        lax.fori_loop(0, NCHUNK, body, 0)
        plsc.subcore_barrier()
        pltpu.sync_copy(aggr_sh.at[pl.ds(s * RPT, RPT)],
                        out_hbm.at[c, pl.ds(s * RPT, RPT)])

        @pl.when(s == NS - 1)
        def _():
            pltpu.sync_copy(aggr_sh.at[pl.ds(NS * RPT, TAIL)],
                            out_hbm.at[c, pl.ds(NS * RPT, TAIL)])

    return agg(h, e3, zeros)


def _mlp_body(relu_out, h_ref, p_ref, w1_ref, b1_ref, g1_ref, be1_ref,
              w2_ref, b2_ref, g2_ref, be2_ref, o_ref):
    out = h_ref[...] + p_ref[0] + p_ref[1]
    z = jnp.dot(out, w1_ref[...], preferred_element_type=jnp.float32)
    z = z + b1_ref[...]
    m = jnp.mean(z, axis=0, keepdims=True)
    v = jnp.mean((z - m) ** 2, axis=0, keepdims=True)
    z = (z - m) * lax.rsqrt(v + 1e-5) * g1_ref[...] + be1_ref[...]
    z = jnp.maximum(z, 0.0)
    z = jnp.dot(z, w2_ref[...], preferred_element_type=jnp.float32)
    z = z + b2_ref[...]
    m2 = jnp.mean(z, axis=0, keepdims=True)
    v2 = jnp.mean((z - m2) ** 2, axis=0, keepdims=True)
    z = (z - m2) * lax.rsqrt(v2 + 1e-5) * g2_ref[...] + be2_ref[...]
    if relu_out:
        z = jnp.maximum(z, 0.0)
    o_ref[...] = z


def _mlp(h, parts, w1, b1, g1, be1, w2, b2, g2, be2, relu_out):
    return pl.pallas_call(
        functools.partial(_mlp_body, relu_out),
        out_shape=jax.ShapeDtypeStruct((N, D), jnp.float32),
    )(h, parts, w1, b1.reshape(1, D), g1.reshape(1, D), be1.reshape(1, D),
      w2, b2.reshape(1, D), g2.reshape(1, D), be2.reshape(1, D))


def kernel(x, edge_index, W1, b1, g1, be1, W2, b2, g2, be2):
    e3 = edge_index.astype(jnp.int32).reshape(2, NW * NCHUNK, CHUNK)
    zeros = jnp.zeros((N, D), jnp.float32)
    h = x
    for l in range(L):
        parts = _sc_aggregate(h, e3, zeros)
        h = _mlp(h, parts, W1[l], b1[l], g1[l], be1[l],
                 W2[l], b2[l], g2[l], be2[l], l < L - 1)
    return h


# speedup vs baseline: 10.6176x; 1.4044x over previous
import functools

import jax
import jax.numpy as jnp
from jax import lax
from jax.experimental import pallas as pl
from jax.experimental.pallas import tpu as pltpu
from jax.experimental.pallas import tpu_sc as plsc

N = 10000
D = 128
E = 320000
L = 3

NC = 2
NS = 16
NW = NC * NS
EPW = E // NW
CHUNK = 125
NCHUNK = EPW // CHUNK
G = 16
NGROUP = NCHUNK // G
RPT = 624
TAIL = N - NS * RPT


def _sc_aggregate(h, e3, zeros):
    mesh = plsc.VectorSubcoreMesh(core_axis_name="c", subcore_axis_name="s")

    @functools.partial(
        pl.kernel,
        out_type=jax.ShapeDtypeStruct((NC, N, D), jnp.float32),
        mesh=mesh,
        scratch_types=[
            pltpu.VMEM_SHARED((N, D), jnp.float32),
            pltpu.VMEM((G, CHUNK), jnp.int32),
            pltpu.VMEM((G, CHUNK), jnp.int32),
            pltpu.VMEM((CHUNK, D), jnp.float32),
            pltpu.VMEM((CHUNK, D), jnp.float32),
            pltpu.SemaphoreType.DMA,
            pltpu.SemaphoreType.DMA,
        ],
    )
    def agg(h_hbm, e_hbm, z_hbm, out_hbm, aggr_sh, idx_src, idx_dst,
            rows0, rows1, sem0, sem1):
        c = lax.axis_index("c")
        s = lax.axis_index("s")
        wid = c * NS + s
        pltpu.sync_copy(z_hbm.at[pl.ds(s * RPT, RPT)],
                        aggr_sh.at[pl.ds(s * RPT, RPT)])

        @pl.when(s == NS - 1)
        def _():
            pltpu.sync_copy(z_hbm.at[pl.ds(NS * RPT, TAIL)],
                            aggr_sh.at[pl.ds(NS * RPT, TAIL)])
        plsc.subcore_barrier()

        def group(g, carry):
            base = wid * NCHUNK + g * G
            pltpu.sync_copy(e_hbm.at[0, pl.ds(base, G)], idx_src)
            pltpu.sync_copy(e_hbm.at[1, pl.ds(base, G)], idx_dst)
            pltpu.async_copy(h_hbm.at[idx_src.at[0]], rows0, sem0)

            def body(j, carry2):
                i0 = 2 * j
                pltpu.async_copy(h_hbm.at[idx_src.at[i0 + 1]], rows1, sem1)
                pltpu.make_async_copy(h_hbm.at[idx_src.at[i0]], rows0,
                                      sem0).wait()
                pltpu.sync_copy(rows0, aggr_sh.at[idx_dst.at[i0]], add=True)

                @pl.when(j < G // 2 - 1)
                def _():
                    pltpu.async_copy(h_hbm.at[idx_src.at[i0 + 2]], rows0,
                                     sem0)

                pltpu.make_async_copy(h_hbm.at[idx_src.at[i0 + 1]], rows1,
                                      sem1).wait()
                pltpu.sync_copy(rows1, aggr_sh.at[idx_dst.at[i0 + 1]],
                                add=True)
                return carry2

            lax.fori_loop(0, G // 2, body, 0)
            return carry

        lax.fori_loop(0, NGROUP, group, 0)
        plsc.subcore_barrier()
        pltpu.sync_copy(aggr_sh.at[pl.ds(s * RPT, RPT)],
                        out_hbm.at[c, pl.ds(s * RPT, RPT)])

        @pl.when(s == NS - 1)
        def _():
            pltpu.sync_copy(aggr_sh.at[pl.ds(NS * RPT, TAIL)],
                            out_hbm.at[c, pl.ds(NS * RPT, TAIL)])

    return agg(h, e3, zeros)


def _mlp_body(relu_out, h_ref, p_ref, w1_ref, b1_ref, g1_ref, be1_ref,
              w2_ref, b2_ref, g2_ref, be2_ref, o_ref):
    out = h_ref[...] + p_ref[0] + p_ref[1]
    z = jnp.dot(out, w1_ref[...], preferred_element_type=jnp.float32)
    z = z + b1_ref[...]
    m = jnp.mean(z, axis=0, keepdims=True)
    v = jnp.mean((z - m) ** 2, axis=0, keepdims=True)
    z = (z - m) * lax.rsqrt(v + 1e-5) * g1_ref[...] + be1_ref[...]
    z = jnp.maximum(z, 0.0)
    z = jnp.dot(z, w2_ref[...], preferred_element_type=jnp.float32)
    z = z + b2_ref[...]
    m2 = jnp.mean(z, axis=0, keepdims=True)
    v2 = jnp.mean((z - m2) ** 2, axis=0, keepdims=True)
    z = (z - m2) * lax.rsqrt(v2 + 1e-5) * g2_ref[...] + be2_ref[...]
    if relu_out:
        z = jnp.maximum(z, 0.0)
    o_ref[...] = z


def _mlp(h, parts, w1, b1, g1, be1, w2, b2, g2, be2, relu_out):
    return pl.pallas_call(
        functools.partial(_mlp_body, relu_out),
        out_shape=jax.ShapeDtypeStruct((N, D), jnp.float32),
    )(h, parts, w1, b1.reshape(1, D), g1.reshape(1, D), be1.reshape(1, D),
      w2, b2.reshape(1, D), g2.reshape(1, D), be2.reshape(1, D))


def kernel(x, edge_index, W1, b1, g1, be1, W2, b2, g2, be2):
    e3 = edge_index.astype(jnp.int32).reshape(2, NW * NCHUNK, CHUNK)
    zeros = jnp.zeros((N, D), jnp.float32)
    h = x
    for l in range(L):
        parts = _sc_aggregate(h, e3, zeros)
        h = _mlp(h, parts, W1[l], b1[l], g1[l], be1[l],
                 W2[l], b2[l], g2[l], be2[l], l < L - 1)
    return h
